# Initial kernel scaffold; baseline (speedup 1.0000x reference)
import functools

import jax
import jax.numpy as jnp
from jax import lax
from jax.experimental import pallas as pl
from jax.experimental.pallas import tpu as pltpu
from jax.experimental.pallas import tpu_sc as plsc

N = 10000
D_IN = 256
D_OUT = 256
DEG = 16
E = N * DEG
L = 16
NC = 2
NS = 16
NW = NC * NS
NPT = -(-N // NW)
NCH = D_IN // L

_mesh = plsc.VectorSubcoreMesh(core_axis_name="c", subcore_axis_name="s")


def _i16(v):
    return jnp.full((L,), v, dtype=jnp.int32)


def _rsqrt(s):
    i = plsc.bitcast(s, jnp.int32)
    i = jnp.int32(0x5F3759DF) - (i >> 1)
    y = plsc.bitcast(i, jnp.float32)
    for _ in range(3):
        y = y * (jnp.float32(1.5) - jnp.float32(0.5) * s * y * y)
    return y


@functools.partial(
    pl.kernel,
    out_type=jax.ShapeDtypeStruct((N, 4 * D_IN), jnp.float32),
    mesh=_mesh,
    scratch_types=[
        pltpu.VMEM((N, 3), jnp.float32),
        pltpu.VMEM((N, 3), jnp.float32),
        pltpu.VMEM((NPT * DEG,), jnp.int32),
        pltpu.VMEM((DEG, D_IN), jnp.float32),
        pltpu.VMEM((4, DEG), jnp.float32),
        pltpu.VMEM((4 * D_IN,), jnp.float32),
        pltpu.VMEM((L,), jnp.float32),
        pltpu.SemaphoreType.DMA,
    ],
)
def _sc_edge_stage(feats_hbm, ipos_hbm, opos_hbm, ext_hbm, nidx_hbm, agg_hbm,
                   ipos_v, opos_v, idx_v, rows_v, w_v, acc_v, ext_v, sem):
    wid = lax.axis_index("s") * NC + lax.axis_index("c")
    base = wid * NPT
    count = jnp.minimum(NPT, N - base)

    pltpu.sync_copy(ipos_hbm, ipos_v)
    pltpu.sync_copy(opos_hbm, opos_v)
    pltpu.sync_copy(ext_hbm, ext_v)
    pltpu.sync_copy(nidx_hbm.at[pl.ds(base * DEG, NPT * DEG)], idx_v)

    ext = ext_v[...]
    c0 = _i16(0)
    c1 = _i16(1)
    c2 = _i16(2)

    def body(n, _):
        node = base + n
        idx = idx_v[pl.ds(n * DEG, DEG)]
        cp = pltpu.async_copy(feats_hbm.at[idx], rows_v, sem)

        nodev = jnp.full((L,), node, dtype=jnp.int32)
        px = plsc.load_gather(ipos_v, [idx, c0])
        py = plsc.load_gather(ipos_v, [idx, c1])
        pz = plsc.load_gather(ipos_v, [idx, c2])
        qx = plsc.load_gather(opos_v, [nodev, c0])
        qy = plsc.load_gather(opos_v, [nodev, c1])
        qz = plsc.load_gather(opos_v, [nodev, c2])
        dx = px - qx
        dy = py - qy
        dz = pz - qz
        s = dx * dx + dy * dy + dz * dz
        rinv = _rsqrt(s)
        r = s * rinv
        w_v[0, :] = r / ext
        w_v[1, :] = dx * rinv
        w_v[2, :] = dz * rinv
        w_v[3, :] = dy * rinv

        cp.wait()

        for x in range(4):
            xv = _i16(x)
            wk = [plsc.load_gather(w_v, [xv, _i16(k)]) for k in range(DEG)]
            for c in range(NCH):
                acc = wk[0] * rows_v[0, pl.ds(c * L, L)]
                for k in range(1, DEG):
                    acc = acc + wk[k] * rows_v[k, pl.ds(c * L, L)]
                acc_v[pl.ds(x * D_IN + c * L, L)] = acc
        pltpu.sync_copy(acc_v, agg_hbm.at[node])
        return 0

    lax.fori_loop(0, count, body, 0)


def _mm_body(x_ref, w_ref, o_ref):
    o_ref[...] = jnp.dot(x_ref[...], w_ref[...],
                         preferred_element_type=jnp.float32)


def _matmul(agg, kflat):
    bm = 1000
    return pl.pallas_call(
        _mm_body,
        grid=(N // bm,),
        in_specs=[
            pl.BlockSpec((bm, 4 * D_IN), lambda i: (i, 0)),
            pl.BlockSpec((4 * D_IN, D_OUT), lambda i: (0, 0)),
        ],
        out_specs=pl.BlockSpec((bm, D_OUT), lambda i: (i, 0)),
        out_shape=jax.ShapeDtypeStruct((N, D_OUT), jnp.float32),
    )(agg, kflat)


def kernel(inp_features, inp_positions, out_positions, extents,
           neighbors_index, neighbors_row_splits, kernel):
    del neighbors_row_splits
    nidx = neighbors_index.astype(jnp.int32)
    nidx = jnp.pad(nidx, (0, NW * NPT * DEG - E))
    ext16 = jnp.broadcast_to(extents.astype(jnp.float32), (L,))
    agg = _sc_edge_stage(inp_features, inp_positions, out_positions,
                         ext16, nidx)
    kflat = kernel.reshape(4 * D_IN, D_OUT)
    return _matmul(agg, kflat)


# speedup vs baseline: 2.2395x; 2.2395x over previous
import functools

import jax
import jax.numpy as jnp
from jax import lax
from jax.experimental import pallas as pl
from jax.experimental.pallas import tpu as pltpu
from jax.experimental.pallas import tpu_sc as plsc

N = 10000
D_IN = 256
D_OUT = 256
DEG = 16
E = N * DEG
L = 16
NC = 2
NS = 16
NW = NC * NS
NPT = -(-N // NW)
NCH = D_IN // L

_mesh = plsc.VectorSubcoreMesh(core_axis_name="c", subcore_axis_name="s")


def _i16(v):
    return jnp.full((L,), v, dtype=jnp.int32)


def _rsqrt(s):
    i = plsc.bitcast(s, jnp.int32)
    i = jnp.int32(0x5F3759DF) - (i >> 1)
    y = plsc.bitcast(i, jnp.float32)
    for _ in range(3):
        y = y * (jnp.float32(1.5) - jnp.float32(0.5) * s * y * y)
    return y


@functools.partial(
    pl.kernel,
    out_type=jax.ShapeDtypeStruct((N, 4 * D_IN), jnp.float32),
    mesh=_mesh,
    compiler_params=pltpu.CompilerParams(needs_layout_passes=False),
    scratch_types=[
        pltpu.VMEM((3 * N,), jnp.float32),
        pltpu.VMEM((3 * N,), jnp.float32),
        pltpu.VMEM((NPT * DEG,), jnp.int32),
        pltpu.VMEM((DEG, D_IN), jnp.float32),
        pltpu.VMEM((2 * 4 * D_IN,), jnp.float32),
        pltpu.VMEM((L,), jnp.float32),
        pltpu.SemaphoreType.DMA,
    ],
)
def _sc_edge_stage(feats_hbm, ipos_hbm, opos_hbm, ext_hbm, nidx_hbm, agg_hbm,
                   ipos_v, opos_v, idx_v, rows_v, acc_v, ext_v, sem):
    wid = lax.axis_index("s") * NC + lax.axis_index("c")
    base = wid * NPT
    count = jnp.minimum(NPT, N - base)

    pltpu.sync_copy(ipos_hbm, ipos_v)
    pltpu.sync_copy(opos_hbm, opos_v)
    pltpu.sync_copy(ext_hbm, ext_v)
    pltpu.sync_copy(nidx_hbm.at[pl.ds(base * DEG, NPT * DEG)], idx_v)

    ext = ext_v[...]
    c0 = _i16(0)
    c1 = _i16(1)
    c2 = _i16(2)

    def body(n, _):
        node = base + n
        idx = idx_v[pl.ds(n * DEG, DEG)]
        cp = pltpu.async_copy(feats_hbm.at[idx_v.at[pl.ds(n * DEG, DEG)]],
                              rows_v, sem)

        idx3 = idx * 3
        node3 = jnp.full((L,), 3 * node, dtype=jnp.int32)
        px = plsc.load_gather(ipos_v, [idx3 + c0])
        py = plsc.load_gather(ipos_v, [idx3 + c1])
        pz = plsc.load_gather(ipos_v, [idx3 + c2])
        qx = plsc.load_gather(opos_v, [node3 + c0])
        qy = plsc.load_gather(opos_v, [node3 + c1])
        qz = plsc.load_gather(opos_v, [node3 + c2])
        dx = px - qx
        dy = py - qy
        dz = pz - qz
        s = dx * dx + dy * dy + dz * dz
        rinv = _rsqrt(s)
        r = s * rinv
        w = [r / ext,
             dx * rinv,
             dz * rinv,
             dy * rinv]

        cp.wait()

        ob = (n & 1) * (4 * D_IN)
        for x in range(4):
            wk = [jnp.full((L,), w[x][k]) for k in range(DEG)]
            for c in range(NCH):
                acc = wk[0] * rows_v[0, pl.ds(c * L, L)]
                for k in range(1, DEG):
                    acc = acc + wk[k] * rows_v[k, pl.ds(c * L, L)]
                acc_v[pl.ds(ob + x * D_IN + c * L, L)] = acc
        pltpu.sync_copy(acc_v.at[pl.ds(ob, 4 * D_IN)], agg_hbm.at[node])
        return 0

    lax.fori_loop(0, count, body, 0)


def _mm_body(x_ref, w_ref, o_ref):
    o_ref[...] = jnp.dot(x_ref[...], w_ref[...],
                         precision=lax.Precision.HIGHEST,
                         preferred_element_type=jnp.float32)


def _matmul(agg, kflat):
    bm = 1000
    return pl.pallas_call(
        _mm_body,
        grid=(N // bm,),
        in_specs=[
            pl.BlockSpec((bm, 4 * D_IN), lambda i: (i, 0)),
            pl.BlockSpec((4 * D_IN, D_OUT), lambda i: (0, 0)),
        ],
        out_specs=pl.BlockSpec((bm, D_OUT), lambda i: (i, 0)),
        out_shape=jax.ShapeDtypeStruct((N, D_OUT), jnp.float32),
    )(agg, kflat)


def kernel(inp_features, inp_positions, out_positions, extents,
           neighbors_index, neighbors_row_splits, kernel):
    del neighbors_row_splits
    nidx = neighbors_index.astype(jnp.int32)
    nidx = jnp.pad(nidx, (0, NW * NPT * DEG - E))
    ext16 = jnp.broadcast_to(extents.astype(jnp.float32), (L,))
    agg = _sc_edge_stage(inp_features, inp_positions.reshape(-1),
                         out_positions.reshape(-1), ext16, nidx)
    kflat = kernel.reshape(4 * D_IN, D_OUT)
    return _matmul(agg, kflat)


# speedup vs baseline: 6.7548x; 3.0162x over previous
import functools

import jax
import jax.numpy as jnp
from jax import lax
from jax.experimental import pallas as pl
from jax.experimental.pallas import tpu as pltpu
from jax.experimental.pallas import tpu_sc as plsc

N = 10000
D_IN = 256
D_OUT = 256
DEG = 16
E = N * DEG
L = 16
NC = 2
NS = 16
NW = NC * NS
NPT = -(-N // NW)
NCH = D_IN // L

_mesh = plsc.VectorSubcoreMesh(core_axis_name="c", subcore_axis_name="s")


def _i16(v):
    return jnp.full((L,), v, dtype=jnp.int32)


def _rsqrt(s):
    i = plsc.bitcast(s, jnp.int32)
    i = jnp.int32(0x5F3759DF) - (i >> 1)
    y = plsc.bitcast(i, jnp.float32)
    for _ in range(3):
        y = y * (jnp.float32(1.5) - jnp.float32(0.5) * s * y * y)
    return y


T = (NPT + 1) // 2
IDXN = NPT + 3


@functools.partial(
    pl.kernel,
    out_type=jax.ShapeDtypeStruct((N, 4 * D_IN), jnp.float32),
    mesh=_mesh,
    compiler_params=pltpu.CompilerParams(needs_layout_passes=False),
    scratch_types=[
        pltpu.VMEM((3 * N,), jnp.float32),
        pltpu.VMEM((3 * N,), jnp.float32),
        pltpu.VMEM((IDXN * DEG,), jnp.int32),
        pltpu.VMEM((2, DEG, D_IN), jnp.float32),
        pltpu.VMEM((2 * 4 * D_IN,), jnp.float32),
        pltpu.VMEM((L,), jnp.float32),
        pltpu.SemaphoreType.DMA,
        pltpu.SemaphoreType.DMA,
    ],
)
def _sc_edge_stage(feats_hbm, ipos_hbm, opos_hbm, ext_hbm, nidx_hbm, agg_hbm,
                   ipos_v, opos_v, idx_v, rows_v, acc_v, ext_v, semA, semB):
    wid = lax.axis_index("s") * NC + lax.axis_index("c")
    base = wid * NPT
    count = jnp.minimum(NPT, N - base)

    pltpu.sync_copy(ipos_hbm, ipos_v)
    pltpu.sync_copy(opos_hbm, opos_v)
    pltpu.sync_copy(ext_hbm, ext_v)
    pltpu.sync_copy(nidx_hbm.at[pl.ds(base * DEG, IDXN * DEG)], idx_v)

    ext = ext_v[...]
    c0 = _i16(0)
    c1 = _i16(1)
    c2 = _i16(2)

    def issue(m, b, sem):
        pltpu.async_copy(feats_hbm.at[idx_v.at[pl.ds(m * DEG, DEG)]],
                         rows_v.at[b], sem)

    def wait(m, b, sem):
        pltpu.make_async_copy(feats_hbm.at[idx_v.at[pl.ds(m * DEG, DEG)]],
                              rows_v.at[b], sem).wait()

    def polar(n, node):
        idx = idx_v[pl.ds(n * DEG, DEG)]
        idx3 = idx * 3
        nq = jnp.minimum(node, N - 1)
        node3 = jnp.full((L,), 3 * nq, dtype=jnp.int32)
        px = plsc.load_gather(ipos_v, [idx3 + c0])
        py = plsc.load_gather(ipos_v, [idx3 + c1])
        pz = plsc.load_gather(ipos_v, [idx3 + c2])
        qx = plsc.load_gather(opos_v, [node3 + c0])
        qy = plsc.load_gather(opos_v, [node3 + c1])
        qz = plsc.load_gather(opos_v, [node3 + c2])
        dx = px - qx
        dy = py - qy
        dz = pz - qz
        s = dx * dx + dy * dy + dz * dz
        rinv = _rsqrt(s)
        r = s * rinv
        return [r / ext,
                dx * rinv,
                dz * rinv,
                dy * rinv]

    def accumulate(w, b):
        ob = b * (4 * D_IN)
        for h in range(2):
            w0k = [jnp.full((L,), w[2 * h][k]) for k in range(DEG)]
            w1k = [jnp.full((L,), w[2 * h + 1][k]) for k in range(DEG)]
            for c in range(NCH):
                row = rows_v[b, 0, pl.ds(c * L, L)]
                a0 = w0k[0] * row
                a1 = w1k[0] * row
                for k in range(1, DEG):
                    row = rows_v[b, k, pl.ds(c * L, L)]
                    a0 = a0 + w0k[k] * row
                    a1 = a1 + w1k[k] * row
                acc_v[pl.ds(ob + (2 * h) * D_IN + c * L, L)] = a0
                acc_v[pl.ds(ob + (2 * h + 1) * D_IN + c * L, L)] = a1

    def flush(node, b):
        @pl.when(node < count)
        def _():
            pltpu.sync_copy(acc_v.at[pl.ds(b * (4 * D_IN), 4 * D_IN)],
                            agg_hbm.at[base + node])

    issue(0, 0, semA)

    def body(t, _):
        n0 = 2 * t
        n1 = n0 + 1
        issue(n1, 1, semB)
        wait(n0, 0, semA)
        w0 = polar(n0, base + n0)
        accumulate(w0, 0)
        flush(n0, 0)

        @pl.when(t < T - 1)
        def _():
            issue(n1 + 1, 0, semA)
        wait(n1, 1, semB)
        w1 = polar(n1, base + n1)
        accumulate(w1, 1)
        flush(n1, 1)
        return 0

    lax.fori_loop(0, T, body, 0)


def _mm_body(x_ref, w_ref, o_ref):
    o_ref[...] = jnp.dot(x_ref[...], w_ref[...],
                         precision=lax.Precision.HIGHEST,
                         preferred_element_type=jnp.float32)


def _matmul(agg, kflat):
    bm = 1000
    return pl.pallas_call(
        _mm_body,
        grid=(N // bm,),
        in_specs=[
            pl.BlockSpec((bm, 4 * D_IN), lambda i: (i, 0)),
            pl.BlockSpec((4 * D_IN, D_OUT), lambda i: (0, 0)),
        ],
        out_specs=pl.BlockSpec((bm, D_OUT), lambda i: (i, 0)),
        out_shape=jax.ShapeDtypeStruct((N, D_OUT), jnp.float32),
    )(agg, kflat)


def kernel(inp_features, inp_positions, out_positions, extents,
           neighbors_index, neighbors_row_splits, kernel):
    del neighbors_row_splits
    nidx = neighbors_index.astype(jnp.int32)
    nidx = jnp.pad(nidx, (0, (NW * NPT + IDXN) * DEG - E))
    ext16 = jnp.broadcast_to(extents.astype(jnp.float32), (L,))
    agg = _sc_edge_stage(inp_features, inp_positions.reshape(-1),
                         out_positions.reshape(-1), ext16, nidx)
    kflat = kernel.reshape(4 * D_IN, D_OUT)
    return _matmul(agg, kflat)


# speedup vs baseline: 6.8555x; 1.0149x over previous
import functools

import jax
import jax.numpy as jnp
from jax import lax
from jax.experimental import pallas as pl
from jax.experimental.pallas import tpu as pltpu
from jax.experimental.pallas import tpu_sc as plsc

N = 10000
D_IN = 256
D_OUT = 256
DEG = 16
E = N * DEG
L = 16
NC = 2
NS = 16
NW = NC * NS
NPT = -(-N // NW)
NCH = D_IN // L

_mesh = plsc.VectorSubcoreMesh(core_axis_name="c", subcore_axis_name="s")


def _i16(v):
    return jnp.full((L,), v, dtype=jnp.int32)


def _rsqrt(s):
    i = plsc.bitcast(s, jnp.int32)
    i = jnp.int32(0x5F3759DF) - (i >> 1)
    y = plsc.bitcast(i, jnp.float32)
    for _ in range(3):
        y = y * (jnp.float32(1.5) - jnp.float32(0.5) * s * y * y)
    return y


T = (NPT + 1) // 2
IDXN = NPT + 3


@functools.partial(
    pl.kernel,
    out_type=jax.ShapeDtypeStruct((N + NW, 4 * D_IN), jnp.float32),
    mesh=_mesh,
    compiler_params=pltpu.CompilerParams(needs_layout_passes=False),
    scratch_types=[
        pltpu.VMEM((3 * N,), jnp.float32),
        pltpu.VMEM((3 * N,), jnp.float32),
        pltpu.VMEM((IDXN * DEG,), jnp.int32),
        pltpu.VMEM((2, DEG, D_IN), jnp.float32),
        pltpu.VMEM((2 * 4 * D_IN,), jnp.float32),
        pltpu.VMEM((L,), jnp.float32),
        pltpu.SemaphoreType.DMA,
        pltpu.SemaphoreType.DMA,
        pltpu.SemaphoreType.DMA,
        pltpu.SemaphoreType.DMA,
    ],
)
def _sc_edge_stage(feats_hbm, ipos_hbm, opos_hbm, ext_hbm, nidx_hbm, agg_hbm,
                   ipos_v, opos_v, idx_v, rows_v, acc_v, ext_v,
                   semA, semB, semOA, semOB):
    wid = lax.axis_index("s") * NC + lax.axis_index("c")
    base = wid * NPT
    count = jnp.minimum(NPT, N - base)

    pltpu.sync_copy(ipos_hbm, ipos_v)
    pltpu.sync_copy(opos_hbm, opos_v)
    pltpu.sync_copy(ext_hbm, ext_v)
    pltpu.sync_copy(nidx_hbm.at[pl.ds(base * DEG, IDXN * DEG)], idx_v)

    ext = ext_v[...]
    c0 = _i16(0)
    c1 = _i16(1)
    c2 = _i16(2)

    def issue(m, b, sem):
        pltpu.async_copy(feats_hbm.at[idx_v.at[pl.ds(m * DEG, DEG)]],
                         rows_v.at[b], sem)

    def wait(m, b, sem):
        pltpu.make_async_copy(feats_hbm.at[idx_v.at[pl.ds(m * DEG, DEG)]],
                              rows_v.at[b], sem).wait()

    def polar(n, node):
        idx = idx_v[pl.ds(n * DEG, DEG)]
        idx3 = idx * 3
        nq = jnp.minimum(node, N - 1)
        node3 = jnp.full((L,), 3 * nq, dtype=jnp.int32)
        px = plsc.load_gather(ipos_v, [idx3 + c0])
        py = plsc.load_gather(ipos_v, [idx3 + c1])
        pz = plsc.load_gather(ipos_v, [idx3 + c2])
        qx = plsc.load_gather(opos_v, [node3 + c0])
        qy = plsc.load_gather(opos_v, [node3 + c1])
        qz = plsc.load_gather(opos_v, [node3 + c2])
        dx = px - qx
        dy = py - qy
        dz = pz - qz
        s = dx * dx + dy * dy + dz * dz
        rinv = _rsqrt(s)
        r = s * rinv
        return [r / ext,
                dx * rinv,
                dz * rinv,
                dy * rinv]

    def accumulate(w, b):
        ob = b * (4 * D_IN)
        for h in range(2):
            w0k = [jnp.full((L,), w[2 * h][k]) for k in range(DEG)]
            w1k = [jnp.full((L,), w[2 * h + 1][k]) for k in range(DEG)]
            for c in range(NCH):
                row = rows_v[b, 0, pl.ds(c * L, L)]
                a0 = w0k[0] * row
                a1 = w1k[0] * row
                for k in range(1, DEG):
                    row = rows_v[b, k, pl.ds(c * L, L)]
                    a0 = a0 + w0k[k] * row
                    a1 = a1 + w1k[k] * row
                acc_v[pl.ds(ob + (2 * h) * D_IN + c * L, L)] = a0
                acc_v[pl.ds(ob + (2 * h + 1) * D_IN + c * L, L)] = a1

    def out_slice(b):
        return acc_v.at[pl.ds(b * (4 * D_IN), 4 * D_IN)]

    def out_dst(n):
        return agg_hbm.at[jnp.where(n < count, base + n, N + wid)]

    def wait_out(n, b, semO, t):
        @pl.when(t > 0)
        def _():
            pltpu.make_async_copy(out_slice(b), out_dst(n), semO).wait()

    issue(0, 0, semA)

    def body(t, _):
        n0 = 2 * t
        n1 = n0 + 1
        issue(n1, 1, semB)
        wait(n0, 0, semA)
        w0 = polar(n0, base + n0)
        wait_out(n0, 0, semOA, t)
        accumulate(w0, 0)
        pltpu.async_copy(out_slice(0), out_dst(n0), semOA)

        @pl.when(t < T - 1)
        def _():
            issue(n1 + 1, 0, semA)
        wait(n1, 1, semB)
        w1 = polar(n1, base + n1)
        wait_out(n1, 1, semOB, t)
        accumulate(w1, 1)
        pltpu.async_copy(out_slice(1), out_dst(n1), semOB)
        return 0

    lax.fori_loop(0, T, body, 0)
    pltpu.make_async_copy(out_slice(0), out_dst(0), semOA).wait()
    pltpu.make_async_copy(out_slice(1), out_dst(1), semOB).wait()


def _mm_body(x_ref, w_ref, o_ref):
    o_ref[...] = jnp.dot(x_ref[...], w_ref[...],
                         precision=lax.Precision.HIGHEST,
                         preferred_element_type=jnp.float32)


def _matmul(agg, kflat):
    bm = 1000
    return pl.pallas_call(
        _mm_body,
        grid=(N // bm,),
        in_specs=[
            pl.BlockSpec((bm, 4 * D_IN), lambda i: (i, 0)),
            pl.BlockSpec((4 * D_IN, D_OUT), lambda i: (0, 0)),
        ],
        out_specs=pl.BlockSpec((bm, D_OUT), lambda i: (i, 0)),
        out_shape=jax.ShapeDtypeStruct((N, D_OUT), jnp.float32),
    )(agg, kflat)


def kernel(inp_features, inp_positions, out_positions, extents,
           neighbors_index, neighbors_row_splits, kernel):
    del neighbors_row_splits
    nidx = neighbors_index.astype(jnp.int32)
    nidx = jnp.pad(nidx, (0, (NW * NPT + IDXN) * DEG - E))
    ext16 = jnp.broadcast_to(extents.astype(jnp.float32), (L,))
    agg = _sc_edge_stage(inp_features, inp_positions.reshape(-1),
                         out_positions.reshape(-1), ext16, nidx)
    kflat = kernel.reshape(4 * D_IN, D_OUT)
    return _matmul(agg, kflat)
